# baseline (speedup 1.0000x reference)
import functools

import jax
import jax.numpy as jnp
from jax import lax
from jax.experimental import pallas as pl
from jax.experimental.pallas import tpu as pltpu
from jax.experimental.pallas import tpu_sc as plsc

B = 16384
D = 64
MARGIN = 1.0
NC = 2
NS = 16
NW = NC * NS
PER_W = B // NW
CHUNK = 128
NCHUNK = PER_W // CHUNK
GROUPS = CHUNK // 16


def _sc_body(ent_hbm, rel_hbm, il_hbm, ir_hbm, irel_hbm, inl_hbm, inr_hbm,
             o1_hbm, o2_hbm, o3_hbm,
             il_v, ir_v, irel_v, inl_v, inr_v,
             l_v, r_v, rl_v, nl_v, nr_v,
             s1_v, s2_v, s3_v, sem):
    wid = lax.axis_index("c") * NS + lax.axis_index("s")
    base = wid * PER_W
    iota16 = lax.iota(jnp.int32, 16)

    for c in range(NCHUNK):
        off = base + c * CHUNK
        pltpu.sync_copy(il_hbm.at[pl.ds(off, CHUNK)], il_v)
        pltpu.sync_copy(ir_hbm.at[pl.ds(off, CHUNK)], ir_v)
        pltpu.sync_copy(irel_hbm.at[pl.ds(off, CHUNK)], irel_v)
        pltpu.sync_copy(inl_hbm.at[pl.ds(off, CHUNK)], inl_v)
        pltpu.sync_copy(inr_hbm.at[pl.ds(off, CHUNK)], inr_v)

        h1 = pltpu.async_copy(ent_hbm.at[il_v], l_v, sem)
        h2 = pltpu.async_copy(ent_hbm.at[ir_v], r_v, sem)
        h3 = pltpu.async_copy(rel_hbm.at[irel_v], rl_v, sem)
        h4 = pltpu.async_copy(ent_hbm.at[inl_v], nl_v, sem)
        h5 = pltpu.async_copy(ent_hbm.at[inr_v], nr_v, sem)
        h1.wait(); h2.wait(); h3.wait(); h4.wait(); h5.wait()

        for g in range(GROUPS):
            rvec = iota16 + (g * 16)
            zero = jnp.zeros((16,), jnp.float32)

            def body(dd, accs):
                a1, a2, a3 = accs
                cvec = jnp.full((16,), dd, jnp.int32)
                le = plsc.load_gather(l_v, [rvec, cvec])
                ri = plsc.load_gather(r_v, [rvec, cvec])
                re = plsc.load_gather(rl_v, [rvec, cvec])
                nl = plsc.load_gather(nl_v, [rvec, cvec])
                nr = plsc.load_gather(nr_v, [rvec, cvec])
                a = le + re
                t1 = a - ri
                t2 = (nl + re) - ri
                t3 = a - nr
                return (a1 + t1 * t1, a2 + t2 * t2, a3 + t3 * t3)

            acc1, acc2, acc3 = lax.fori_loop(0, D, body, (zero, zero, zero))
            s1_v[pl.ds(g * 16, 16)] = acc1
            s2_v[pl.ds(g * 16, 16)] = acc2
            s3_v[pl.ds(g * 16, 16)] = acc3

        pltpu.sync_copy(s1_v, o1_hbm.at[pl.ds(off, CHUNK)])
        pltpu.sync_copy(s2_v, o2_hbm.at[pl.ds(off, CHUNK)])
        pltpu.sync_copy(s3_v, o3_hbm.at[pl.ds(off, CHUNK)])


_sc_kernel = functools.partial(
    pl.kernel,
    mesh=plsc.VectorSubcoreMesh(core_axis_name="c", subcore_axis_name="s",
                                num_cores=NC, num_subcores=NS),
    out_type=[jax.ShapeDtypeStruct((B,), jnp.float32)] * 3,
    scratch_types=[
        pltpu.VMEM((CHUNK,), jnp.int32),
        pltpu.VMEM((CHUNK,), jnp.int32),
        pltpu.VMEM((CHUNK,), jnp.int32),
        pltpu.VMEM((CHUNK,), jnp.int32),
        pltpu.VMEM((CHUNK,), jnp.int32),
        pltpu.VMEM((CHUNK, D), jnp.float32),
        pltpu.VMEM((CHUNK, D), jnp.float32),
        pltpu.VMEM((CHUNK, D), jnp.float32),
        pltpu.VMEM((CHUNK, D), jnp.float32),
        pltpu.VMEM((CHUNK, D), jnp.float32),
        pltpu.VMEM((CHUNK,), jnp.float32),
        pltpu.VMEM((CHUNK,), jnp.float32),
        pltpu.VMEM((CHUNK,), jnp.float32),
        pltpu.SemaphoreType.DMA,
    ],
    compiler_params=pltpu.CompilerParams(needs_layout_passes=False,
                                         use_tc_tiling_on_sc=False),
)(_sc_body)


def _tc_body(p_ref, n1_ref, n2_ref, o_ref):
    p = jnp.sqrt(p_ref[...])
    n1 = jnp.sqrt(n1_ref[...])
    n2 = jnp.sqrt(n2_ref[...])
    c1 = p - n1 + MARGIN
    c2 = p - n2 + MARGIN
    cost = c1 * (c1 > 0) + c2 * (c2 > 0)
    o_ref[0, 0] = jnp.sum(cost) * (1.0 / B)


def kernel(entity_table, relation_table, leftEnIndices, rightEnIndices,
           relIndices, negLeftEnIndices, negRightEnIndices):
    il = leftEnIndices.astype(jnp.int32)
    ir = rightEnIndices.astype(jnp.int32)
    irel = relIndices.astype(jnp.int32)
    inl = negLeftEnIndices.astype(jnp.int32)
    inr = negRightEnIndices.astype(jnp.int32)

    s1, s2, s3 = _sc_kernel(entity_table, relation_table, il, ir, irel, inl, inr)

    out = pl.pallas_call(
        _tc_body,
        out_shape=jax.ShapeDtypeStruct((1, 1), jnp.float32),
        out_specs=pl.BlockSpec(memory_space=pltpu.SMEM),
    )(s1.reshape(128, 128), s2.reshape(128, 128), s3.reshape(128, 128))
    return out[0, 0]
